# baseline (speedup 1.0000x reference)
import jax
import jax.numpy as jnp
from jax.experimental import pallas as pl

_N_TYPES = 100000
_SEQ_LEN = 200
_DBLK = 1024


def _bow_block_kernel(in_ref, out_ref):
    j = pl.program_id(0)
    tokcol = in_ref[0:_SEQ_LEN, :]
    val = in_ref[_SEQ_LEN:2 * _SEQ_LEN, :]
    mask = tokcol == (j * _DBLK).astype(jnp.float32)
    out_ref[:, :] = jnp.where(mask, val, 0.0)


def kernel(tokens, vals):
    tokcol = (
        tokens.astype(jnp.int32)[:, None, None]
        - jnp.arange(_DBLK, dtype=jnp.int32)[None, None, :]
    )
    tokcol_f = tokcol.astype(jnp.float32)
    val2 = jnp.broadcast_to(vals[:, None, None], (_SEQ_LEN, 1, _DBLK))
    fused = jnp.concatenate([tokcol_f, val2], axis=0)
    grid = (pl.cdiv(_N_TYPES, _DBLK),)
    out = pl.pallas_call(
        _bow_block_kernel,
        grid=grid,
        in_specs=[
            pl.BlockSpec((2 * _SEQ_LEN, None, _DBLK), lambda j: (0, 0, 0)),
        ],
        out_specs=pl.BlockSpec((_SEQ_LEN, None, _DBLK), lambda j: (0, 0, j)),
        out_shape=jax.ShapeDtypeStruct((_SEQ_LEN, 1, _N_TYPES), jnp.float32),
    )(fused)
    return out


# speedup vs baseline: 1.3194x; 1.3194x over previous
import jax
import jax.numpy as jnp
from jax.experimental import pallas as pl

_N_TYPES = 100000
_SEQ_LEN = 200
_DBLK = 2048


def _bow_block_kernel(in_ref, out_ref):
    j = pl.program_id(0)
    tokcol = in_ref[0:_SEQ_LEN, :]
    val = in_ref[_SEQ_LEN:2 * _SEQ_LEN, :]
    mask = tokcol == (j * _DBLK).astype(jnp.float32)
    out_ref[:, :] = jnp.where(mask, val, 0.0)


def kernel(tokens, vals):
    tokcol = (
        tokens.astype(jnp.int32)[:, None, None]
        - jnp.arange(_DBLK, dtype=jnp.int32)[None, None, :]
    )
    tokcol_f = tokcol.astype(jnp.float32)
    val2 = jnp.broadcast_to(vals[:, None, None], (_SEQ_LEN, 1, _DBLK))
    fused = jnp.concatenate([tokcol_f, val2], axis=0)
    grid = (pl.cdiv(_N_TYPES, _DBLK),)
    out = pl.pallas_call(
        _bow_block_kernel,
        grid=grid,
        in_specs=[
            pl.BlockSpec((2 * _SEQ_LEN, None, _DBLK), lambda j: (0, 0, 0)),
        ],
        out_specs=pl.BlockSpec((_SEQ_LEN, None, _DBLK), lambda j: (0, 0, j)),
        out_shape=jax.ShapeDtypeStruct((_SEQ_LEN, 1, _N_TYPES), jnp.float32),
    )(fused)
    return out


# speedup vs baseline: 1.4809x; 1.1224x over previous
import jax
import jax.numpy as jnp
from jax.experimental import pallas as pl

_N_TYPES = 100000
_SEQ_LEN = 200
_DBLK = 4096


def _bow_block_kernel(tokcol_ref, val_ref, out_ref):
    j = pl.program_id(0)
    mask = tokcol_ref[:, :] == j * _DBLK
    out_ref[:, :] = jnp.where(mask, val_ref[:, :], 0.0)


def kernel(tokens, vals):
    tokcol = (
        tokens.astype(jnp.int32)[:, None, None]
        - jnp.arange(_DBLK, dtype=jnp.int32)[None, None, :]
    )
    val2 = jnp.broadcast_to(vals[:, None, None], (_SEQ_LEN, 1, _DBLK))
    grid = (pl.cdiv(_N_TYPES, _DBLK),)
    out = pl.pallas_call(
        _bow_block_kernel,
        grid=grid,
        in_specs=[
            pl.BlockSpec((_SEQ_LEN, None, _DBLK), lambda j: (0, 0, 0)),
            pl.BlockSpec((_SEQ_LEN, None, _DBLK), lambda j: (0, 0, 0)),
        ],
        out_specs=pl.BlockSpec((_SEQ_LEN, None, _DBLK), lambda j: (0, 0, j)),
        out_shape=jax.ShapeDtypeStruct((_SEQ_LEN, 1, _N_TYPES), jnp.float32),
    )(tokcol, val2)
    return out
